# baseline (speedup 1.0000x reference)
import functools

import jax
import jax.numpy as jnp
from jax import lax
from jax.experimental import pallas as pl
from jax.experimental.pallas import tpu as pltpu
from jax.experimental.pallas import tpu_sc as plsc

BATCH = 16384
EMBED = 32
NUM_CORES = 2
NUM_SUBCORES = 16
LANES = 16
NUM_WORKERS = NUM_CORES * NUM_SUBCORES
BPW = BATCH // NUM_WORKERS
IDX_CHUNK = 128
NCHUNK = BPW // IDX_CHUNK
GROUPS = BPW // LANES


def _bpr_body(user_table, item_table, u_idx_hbm, i_idx_hbm, n_idx_hbm,
              out_hbm,
              u_idx, i_idx, n_idx, u_rows, i_rows, n_rows, out_v,
              sem_u, sem_i, sem_n):
    wid = lax.axis_index("s") * NUM_CORES + lax.axis_index("c")
    base = wid * BPW

    for c in range(NCHUNK):
        off = base + c * IDX_CHUNK
        pltpu.sync_copy(u_idx_hbm.at[pl.ds(off, IDX_CHUNK)], u_idx.at[c])
        pltpu.sync_copy(i_idx_hbm.at[pl.ds(off, IDX_CHUNK)], i_idx.at[c])
        pltpu.sync_copy(n_idx_hbm.at[pl.ds(off, IDX_CHUNK)], n_idx.at[c])

    copies = []
    for c in range(NCHUNK):
        dst = pl.ds(c * IDX_CHUNK, IDX_CHUNK)
        copies.append(pltpu.async_copy(
            user_table.at[u_idx.at[c]], u_rows.at[dst, :], sem_u))
        copies.append(pltpu.async_copy(
            item_table.at[i_idx.at[c]], i_rows.at[dst, :], sem_i))
        copies.append(pltpu.async_copy(
            item_table.at[n_idx.at[c]], n_rows.at[dst, :], sem_n))
    for cp in copies:
        cp.wait()

    lane_iota = lax.iota(jnp.int32, LANES)

    def group_body(g, _):
        acc = jnp.zeros((LANES,), jnp.float32)
        for j in range(LANES):
            r = g * LANES + j
            u0 = u_rows[r, pl.ds(0, LANES)]
            u1 = u_rows[r, pl.ds(LANES, LANES)]
            i0 = i_rows[r, pl.ds(0, LANES)]
            i1 = i_rows[r, pl.ds(LANES, LANES)]
            n0 = n_rows[r, pl.ds(0, LANES)]
            n1 = n_rows[r, pl.ds(LANES, LANES)]
            t = u0 * (i0 - n0) + u1 * (i1 - n1)
            acc = jnp.where(lane_iota == j, jnp.sum(t), acc)
        prob = 1.0 / (1.0 + jnp.exp(-acc))
        out_v[pl.ds(g * LANES, LANES)] = prob
        return ()

    lax.fori_loop(0, GROUPS, group_body, ())

    pltpu.sync_copy(out_v, out_hbm.at[pl.ds(base, BPW)])


@functools.partial(jax.jit, static_argnames=())
def kernel(user_table, item_table, user_tensor, item_tensor, nega_item_tensor):
    mesh = plsc.VectorSubcoreMesh(core_axis_name="c", subcore_axis_name="s")
    run = pl.kernel(
        _bpr_body,
        out_type=jax.ShapeDtypeStruct((BATCH,), jnp.float32),
        mesh=mesh,
        scratch_types=[
            pltpu.VMEM((NCHUNK, IDX_CHUNK), jnp.int32),
            pltpu.VMEM((NCHUNK, IDX_CHUNK), jnp.int32),
            pltpu.VMEM((NCHUNK, IDX_CHUNK), jnp.int32),
            pltpu.VMEM((BPW, EMBED), jnp.float32),
            pltpu.VMEM((BPW, EMBED), jnp.float32),
            pltpu.VMEM((BPW, EMBED), jnp.float32),
            pltpu.VMEM((BPW,), jnp.float32),
            pltpu.SemaphoreType.DMA,
            pltpu.SemaphoreType.DMA,
            pltpu.SemaphoreType.DMA,
        ],
        compiler_params=pltpu.CompilerParams(
            needs_layout_passes=False, use_tc_tiling_on_sc=False),
    )
    return run(
        user_table,
        item_table,
        user_tensor.astype(jnp.int32),
        item_tensor.astype(jnp.int32),
        nega_item_tensor.astype(jnp.int32),
    )


# speedup vs baseline: 2.3046x; 2.3046x over previous
import functools

import jax
import jax.numpy as jnp
from jax import lax
from jax.experimental import pallas as pl
from jax.experimental.pallas import tpu as pltpu
from jax.experimental.pallas import tpu_sc as plsc

BATCH = 16384
EMBED = 32
ROWS = 1000000
NUM_CORES = 2
NUM_SUBCORES = 16
LANES = 16
NUM_WORKERS = NUM_CORES * NUM_SUBCORES
BPW = BATCH // NUM_WORKERS
GROUPS = BPW // LANES


def _bpr_body(u_tab, i_tab, u_idx_hbm, i_idx_hbm, n_idx_hbm,
              out_hbm,
              vidx, ring, out_v, sem_u, sem_i, sem_n):
    wid = lax.axis_index("s") * NUM_CORES + lax.axis_index("c")
    base = wid * BPW

    idx_hbms = (u_idx_hbm, i_idx_hbm, n_idx_hbm)
    tabs = (u_tab, i_tab, i_tab)
    sems = (sem_u, sem_i, sem_n)

    for t in range(3):
        pltpu.sync_copy(idx_hbms[t].at[pl.ds(base, BPW)],
                        vidx.at[pl.ds(t * BPW, BPW)])

    def issue(r, stage):
        for t in range(3):
            c0 = pl.multiple_of(lax.shift_right_logical(r[t], 7) * 128, 128)
            pltpu.async_copy(tabs[t].at[:, pl.ds(c0, 128)],
                             ring.at[stage, t], sems[t])

    def drain(s):
        for t in range(3):
            pltpu.make_async_copy(tabs[t].at[:, pl.ds(0, 128)],
                                  ring.at[s, t], sems[t]).wait()

    lane_iota = lax.iota(jnp.int32, LANES)
    rows_lo = lane_iota
    rows_hi = lane_iota + LANES

    first = [vidx[pl.ds(t * BPW, LANES)] for t in range(3)]
    issue([v[0] for v in first], 0)

    def group_body(g, _):
        rvec = [vidx[pl.ds(t * BPW + g * LANES, LANES)] for t in range(3)]
        nxt_off = jnp.minimum((g + 1) * LANES, BPW - LANES)
        rnxt = [vidx[pl.ds(t * BPW + nxt_off, LANES)] for t in range(3)]
        acc = jnp.zeros((LANES,), jnp.float32)
        for j in range(LANES):
            s = (g * LANES + j) % 2
            if j < LANES - 1:
                issue([v[j + 1] for v in rvec], 1 - s)
            else:
                issue([v[0] for v in rnxt], 1 - s)
            drain(s)
            vals = []
            for t in range(3):
                c = jnp.full((LANES,), rvec[t][j] & 127, jnp.int32)
                v0 = plsc.load_gather(ring.at[s, t], [rows_lo, c])
                v1 = plsc.load_gather(ring.at[s, t], [rows_hi, c])
                vals.append((v0, v1))
            (u0, u1), (i0, i1), (n0, n1) = vals
            tv = u0 * (i0 - n0) + u1 * (i1 - n1)
            acc = jnp.where(lane_iota == j, jnp.sum(tv), acc)
        prob = 1.0 / (1.0 + jnp.exp(-acc))
        out_v[pl.ds(g * LANES, LANES)] = prob
        return ()

    lax.fori_loop(0, GROUPS, group_body, ())
    drain(0)

    pltpu.sync_copy(out_v, out_hbm.at[pl.ds(base, BPW)])


@jax.jit
def kernel(user_table, item_table, user_tensor, item_tensor, nega_item_tensor):
    mesh = plsc.VectorSubcoreMesh(core_axis_name="c", subcore_axis_name="s")
    run = pl.kernel(
        _bpr_body,
        out_type=jax.ShapeDtypeStruct((BATCH,), jnp.float32),
        mesh=mesh,
        scratch_types=[
            pltpu.VMEM((3 * BPW,), jnp.int32),
            pltpu.VMEM((2, 3, EMBED, 128), jnp.float32),
            pltpu.VMEM((BPW,), jnp.float32),
            pltpu.SemaphoreType.DMA,
            pltpu.SemaphoreType.DMA,
            pltpu.SemaphoreType.DMA,
        ],
        compiler_params=pltpu.CompilerParams(
            needs_layout_passes=False, use_tc_tiling_on_sc=True),
    )
    return run(
        user_table.T,
        item_table.T,
        user_tensor.astype(jnp.int32),
        item_tensor.astype(jnp.int32),
        nega_item_tensor.astype(jnp.int32),
    )


# speedup vs baseline: 2.6552x; 1.1521x over previous
import functools

import jax
import jax.numpy as jnp
from jax import lax
from jax.experimental import pallas as pl
from jax.experimental.pallas import tpu as pltpu
from jax.experimental.pallas import tpu_sc as plsc

BATCH = 16384
EMBED = 32
ROWS = 1000000
NUM_CORES = 2
NUM_SUBCORES = 16
LANES = 16
NUM_WORKERS = NUM_CORES * NUM_SUBCORES
BPW = BATCH // NUM_WORKERS
GROUPS = BPW // LANES


def _bpr_body(u_tab, i_tab, u_idx_hbm, i_idx_hbm, n_idx_hbm,
              out_hbm,
              vidx, ring, out_v, sem_u, sem_i, sem_n):
    wid = lax.axis_index("s") * NUM_CORES + lax.axis_index("c")
    base = wid * BPW

    idx_hbms = (u_idx_hbm, i_idx_hbm, n_idx_hbm)
    tabs = (u_tab, i_tab, i_tab)
    sems = (sem_u, sem_i, sem_n)

    for t in range(3):
        pltpu.sync_copy(idx_hbms[t].at[pl.ds(base, BPW)],
                        vidx.at[pl.ds(t * BPW, BPW)])

    def issue(r, stage):
        for t in range(3):
            c0 = pl.multiple_of(lax.shift_right_logical(r[t], 7) * 128, 128)
            pltpu.async_copy(tabs[t].at[:, pl.ds(c0, 128)],
                             ring.at[stage, t], sems[t])

    def drain(s):
        for t in range(3):
            pltpu.make_async_copy(tabs[t].at[:, pl.ds(0, 128)],
                                  ring.at[s, t], sems[t]).wait()

    lane_iota = lax.iota(jnp.int32, LANES)
    rows_lo = lane_iota
    rows_hi = lane_iota + LANES

    first = [vidx[pl.ds(t * BPW, LANES)] for t in range(3)]
    issue([v[0] for v in first], 0)
    issue([v[1] for v in first], 1)

    def group_body(g, _):
        rvec = [vidx[pl.ds(t * BPW + g * LANES, LANES)] for t in range(3)]
        nxt_off = jnp.minimum((g + 1) * LANES, BPW - LANES)
        rnxt = [vidx[pl.ds(t * BPW + nxt_off, LANES)] for t in range(3)]
        acc = jnp.zeros((LANES,), jnp.float32)
        for j in range(LANES):
            s = (g * LANES + j) % 4
            s2 = (g * LANES + j + 2) % 4
            if j < LANES - 2:
                issue([v[j + 2] for v in rvec], s2)
            else:
                issue([v[j - (LANES - 2)] for v in rnxt], s2)
            drain(s)
            vals = []
            for t in range(3):
                c = jnp.full((LANES,), rvec[t][j] & 127, jnp.int32)
                v0 = plsc.load_gather(ring.at[s, t], [rows_lo, c])
                v1 = plsc.load_gather(ring.at[s, t], [rows_hi, c])
                vals.append((v0, v1))
            (u0, u1), (i0, i1), (n0, n1) = vals
            tv = u0 * (i0 - n0) + u1 * (i1 - n1)
            acc = jnp.where(lane_iota == j, jnp.sum(tv), acc)
        prob = 1.0 / (1.0 + jnp.exp(-acc))
        out_v[pl.ds(g * LANES, LANES)] = prob
        return ()

    lax.fori_loop(0, GROUPS, group_body, ())
    drain(0)
    drain(1)

    pltpu.sync_copy(out_v, out_hbm.at[pl.ds(base, BPW)])


@jax.jit
def kernel(user_table, item_table, user_tensor, item_tensor, nega_item_tensor):
    mesh = plsc.VectorSubcoreMesh(core_axis_name="c", subcore_axis_name="s")
    run = pl.kernel(
        _bpr_body,
        out_type=jax.ShapeDtypeStruct((BATCH,), jnp.float32),
        mesh=mesh,
        scratch_types=[
            pltpu.VMEM((3 * BPW,), jnp.int32),
            pltpu.VMEM((4, 3, EMBED, 128), jnp.float32),
            pltpu.VMEM((BPW,), jnp.float32),
            pltpu.SemaphoreType.DMA,
            pltpu.SemaphoreType.DMA,
            pltpu.SemaphoreType.DMA,
        ],
        compiler_params=pltpu.CompilerParams(
            needs_layout_passes=False, use_tc_tiling_on_sc=True),
    )
    return run(
        user_table.T,
        item_table.T,
        user_tensor.astype(jnp.int32),
        item_tensor.astype(jnp.int32),
        nega_item_tensor.astype(jnp.int32),
    )


# speedup vs baseline: 2.9912x; 1.1266x over previous
import functools

import jax
import jax.numpy as jnp
from jax import lax
from jax.experimental import pallas as pl
from jax.experimental.pallas import tpu as pltpu
from jax.experimental.pallas import tpu_sc as plsc

BATCH = 16384
EMBED = 32
ROWS = 1000000
NUM_CORES = 2
NUM_SUBCORES = 16
LANES = 16
NUM_WORKERS = NUM_CORES * NUM_SUBCORES
BPW = BATCH // NUM_WORKERS
GROUPS = BPW // LANES


def _bpr_body(u_tab, i_tab, u_idx_hbm, i_idx_hbm, n_idx_hbm,
              out_hbm,
              vidx, ring, out_v, sem_u, sem_i, sem_n):
    wid = lax.axis_index("s") * NUM_CORES + lax.axis_index("c")
    base = wid * BPW

    idx_hbms = (u_idx_hbm, i_idx_hbm, n_idx_hbm)
    tabs = (u_tab, i_tab, i_tab)
    sems = (sem_u, sem_i, sem_n)

    for t in range(3):
        pltpu.sync_copy(idx_hbms[t].at[pl.ds(base, BPW)],
                        vidx.at[pl.ds(t * BPW, BPW)])

    def issue(r, stage):
        for t in range(3):
            c0 = pl.multiple_of(lax.shift_right_logical(r[t], 7) * 128, 128)
            pltpu.async_copy(tabs[t].at[:, pl.ds(c0, 128)],
                             ring.at[stage, t], sems[t])

    def drain(s):
        for t in range(3):
            pltpu.make_async_copy(tabs[t].at[:, pl.ds(0, 128)],
                                  ring.at[s, t], sems[t]).wait()

    lane_iota = lax.iota(jnp.int32, LANES)
    rows_lo = lane_iota
    rows_hi = lane_iota + LANES

    first = [vidx[pl.ds(t * BPW, LANES)] for t in range(3)]
    for p in range(4):
        issue([v[p] for v in first], p)

    def group_body(g, _):
        rvec = [vidx[pl.ds(t * BPW + g * LANES, LANES)] for t in range(3)]
        nxt_off = jnp.minimum((g + 1) * LANES, BPW - LANES)
        rnxt = [vidx[pl.ds(t * BPW + nxt_off, LANES)] for t in range(3)]
        acc = jnp.zeros((LANES,), jnp.float32)
        for j in range(LANES):
            s = (g * LANES + j) % 8
            s2 = (g * LANES + j + 4) % 8
            if j < LANES - 4:
                issue([v[j + 4] for v in rvec], s2)
            else:
                issue([v[j - (LANES - 4)] for v in rnxt], s2)
            drain(s)
            vals = []
            for t in range(3):
                c = jnp.full((LANES,), rvec[t][j] & 127, jnp.int32)
                v0 = plsc.load_gather(ring.at[s, t], [rows_lo, c])
                v1 = plsc.load_gather(ring.at[s, t], [rows_hi, c])
                vals.append((v0, v1))
            (u0, u1), (i0, i1), (n0, n1) = vals
            tv = u0 * (i0 - n0) + u1 * (i1 - n1)
            acc = jnp.where(lane_iota == j, jnp.sum(tv), acc)
        prob = 1.0 / (1.0 + jnp.exp(-acc))
        out_v[pl.ds(g * LANES, LANES)] = prob
        return ()

    lax.fori_loop(0, GROUPS, group_body, ())
    for p in range(4):
        drain(p)

    pltpu.sync_copy(out_v, out_hbm.at[pl.ds(base, BPW)])


@jax.jit
def kernel(user_table, item_table, user_tensor, item_tensor, nega_item_tensor):
    mesh = plsc.VectorSubcoreMesh(core_axis_name="c", subcore_axis_name="s")
    run = pl.kernel(
        _bpr_body,
        out_type=jax.ShapeDtypeStruct((BATCH,), jnp.float32),
        mesh=mesh,
        scratch_types=[
            pltpu.VMEM((3 * BPW,), jnp.int32),
            pltpu.VMEM((8, 3, EMBED, 128), jnp.float32),
            pltpu.VMEM((BPW,), jnp.float32),
            pltpu.SemaphoreType.DMA,
            pltpu.SemaphoreType.DMA,
            pltpu.SemaphoreType.DMA,
        ],
        compiler_params=pltpu.CompilerParams(
            needs_layout_passes=False, use_tc_tiling_on_sc=True),
    )
    return run(
        user_table.T,
        item_table.T,
        user_tensor.astype(jnp.int32),
        item_tensor.astype(jnp.int32),
        nega_item_tensor.astype(jnp.int32),
    )


# speedup vs baseline: 3.0438x; 1.0176x over previous
import functools

import jax
import jax.numpy as jnp
from jax import lax
from jax.experimental import pallas as pl
from jax.experimental.pallas import tpu as pltpu
from jax.experimental.pallas import tpu_sc as plsc

BATCH = 16384
EMBED = 32
ROWS = 1000000
NUM_CORES = 2
NUM_SUBCORES = 16
LANES = 16
NUM_WORKERS = NUM_CORES * NUM_SUBCORES
BPW = BATCH // NUM_WORKERS
GROUPS = BPW // LANES


def _bpr_body(u_tab, i_tab, u_idx_hbm, i_idx_hbm, n_idx_hbm,
              out_hbm,
              vidx, ring, out_v, sem_u, sem_i, sem_n):
    wid = lax.axis_index("s") * NUM_CORES + lax.axis_index("c")
    base = wid * BPW

    idx_hbms = (u_idx_hbm, i_idx_hbm, n_idx_hbm)
    tabs = (u_tab, i_tab, i_tab)
    sems = (sem_u, sem_i, sem_n)

    for t in range(3):
        pltpu.sync_copy(idx_hbms[t].at[pl.ds(base, BPW)],
                        vidx.at[pl.ds(t * BPW, BPW)])

    def issue(r, stage):
        for t in range(3):
            c0 = pl.multiple_of(lax.shift_right_logical(r[t], 7) * 128, 128)
            pltpu.async_copy(tabs[t].at[:, pl.ds(c0, 128)],
                             ring.at[stage, t], sems[t])

    def drain(s):
        for t in range(3):
            pltpu.make_async_copy(tabs[t].at[:, pl.ds(0, 128)],
                                  ring.at[s, t], sems[t]).wait()

    lane_iota = lax.iota(jnp.int32, LANES)
    rows_lo = lane_iota
    rows_hi = lane_iota + LANES

    first = [vidx[pl.ds(t * BPW, LANES)] for t in range(3)]
    for p in range(6):
        issue([v[p] for v in first], p)

    def group_body(g, _):
        rvec = [vidx[pl.ds(t * BPW + g * LANES, LANES)] for t in range(3)]
        nxt_off = jnp.minimum((g + 1) * LANES, BPW - LANES)
        rnxt = [vidx[pl.ds(t * BPW + nxt_off, LANES)] for t in range(3)]
        acc = jnp.zeros((LANES,), jnp.float32)
        for j in range(LANES):
            s = (g * LANES + j) % 8
            s2 = (g * LANES + j + 6) % 8
            if j < LANES - 6:
                issue([v[j + 6] for v in rvec], s2)
            else:
                issue([v[j - (LANES - 6)] for v in rnxt], s2)
            drain(s)
            vals = []
            for t in range(3):
                c = jnp.full((LANES,), rvec[t][j] & 127, jnp.int32)
                v0 = plsc.load_gather(ring.at[s, t], [rows_lo, c])
                v1 = plsc.load_gather(ring.at[s, t], [rows_hi, c])
                vals.append((v0, v1))
            (u0, u1), (i0, i1), (n0, n1) = vals
            tv = u0 * (i0 - n0) + u1 * (i1 - n1)
            acc = jnp.where(lane_iota == j, jnp.sum(tv), acc)
        prob = 1.0 / (1.0 + jnp.exp(-acc))
        out_v[pl.ds(g * LANES, LANES)] = prob
        return ()

    lax.fori_loop(0, GROUPS, group_body, ())
    for p in range(6):
        drain(p)

    pltpu.sync_copy(out_v, out_hbm.at[pl.ds(base, BPW)])


@jax.jit
def kernel(user_table, item_table, user_tensor, item_tensor, nega_item_tensor):
    mesh = plsc.VectorSubcoreMesh(core_axis_name="c", subcore_axis_name="s")
    run = pl.kernel(
        _bpr_body,
        out_type=jax.ShapeDtypeStruct((BATCH,), jnp.float32),
        mesh=mesh,
        scratch_types=[
            pltpu.VMEM((3 * BPW,), jnp.int32),
            pltpu.VMEM((8, 3, EMBED, 128), jnp.float32),
            pltpu.VMEM((BPW,), jnp.float32),
            pltpu.SemaphoreType.DMA,
            pltpu.SemaphoreType.DMA,
            pltpu.SemaphoreType.DMA,
        ],
        compiler_params=pltpu.CompilerParams(
            needs_layout_passes=False, use_tc_tiling_on_sc=True),
    )
    return run(
        user_table.T,
        item_table.T,
        user_tensor.astype(jnp.int32),
        item_tensor.astype(jnp.int32),
        nega_item_tensor.astype(jnp.int32),
    )


# speedup vs baseline: 4.3510x; 1.4295x over previous
import functools

import jax
import jax.numpy as jnp
from jax import lax
from jax.experimental import pallas as pl
from jax.experimental.pallas import tpu as pltpu
from jax.experimental.pallas import tpu_sc as plsc

BATCH = 16384
EMBED = 32
ROWS = 1000000
NUM_CORES = 2
NUM_SUBCORES = 16
LANES = 16
NUM_WORKERS = NUM_CORES * NUM_SUBCORES
BPW = BATCH // NUM_WORKERS
TCOLS = (ROWS + 127) // 128
TPW = (TCOLS + NUM_WORKERS - 1) // NUM_WORKERS
NBUCK = 2 * TPW
CAP = 2048
DEPTH = 8
STAG = 32


def _extract_body(u_tab, i_tab, u_idx_hbm, i_idx_hbm, n_idx_hbm,
                  u_emb, i_emb, n_emb,
                  vidx, mly_r, mly_id, srt_r, srt_id, ring, stag,
                  starts, cursor,
                  sem_slab, sem_out):
    wid = lax.axis_index("s") * NUM_CORES + lax.axis_index("c")
    lo = wid * TPW
    hi = jnp.minimum(lo + TPW, TCOLS)

    idx_hbms = (u_idx_hbm, i_idx_hbm, n_idx_hbm)
    embs = (u_emb, i_emb, n_emb)
    lane_iota = lax.iota(jnp.int32, LANES)

    for t in range(3):
        pltpu.sync_copy(idx_hbms[t], vidx.at[pl.ds(t * BATCH, BATCH)])

    nrec = jnp.int32(0)
    for t in range(3):
        def body(i, off, t=t):
            v = vidx[pl.ds(t * BATCH + i * LANES, LANES)]
            tc = lax.shift_right_logical(v, 7)
            mine = (tc >= lo) & (tc < hi)
            pc = jnp.cumsum(mine.astype(jnp.int32))
            pos = off + pc - 1
            k = i * LANES + lane_iota
            plsc.store_scatter(mly_r, [pos], v, mask=mine)
            plsc.store_scatter(mly_id, [pos], k * 4 + t, mask=mine)
            return off + pc[LANES - 1]
        nrec = lax.fori_loop(0, BATCH // LANES, body, nrec)

    def zero_body(b, _):
        cursor[b] = jnp.int32(0)
        return ()
    lax.fori_loop(0, NBUCK, zero_body, ())

    def hist_body(q, _):
        rv = mly_r[pl.ds(q * LANES, LANES)]
        iv = mly_id[pl.ds(q * LANES, LANES)]
        rem = nrec - q * LANES
        bv = jnp.clip((lax.shift_right_logical(rv, 7) - lo) * 2
                      + jnp.minimum(iv & 3, 1), 0, NBUCK - 1)
        for lane in range(LANES):
            @pl.when(rem > lane)
            def _():
                b = bv[lane]
                cursor[b] = cursor[b] + 1
        return ()
    lax.fori_loop(0, (nrec + LANES - 1) // LANES, hist_body, ())

    def scan_body(b, acc):
        c = cursor[b]
        starts[b] = acc
        cursor[b] = acc
        return acc + c
    total = lax.fori_loop(0, NBUCK, scan_body, jnp.int32(0))
    starts[NBUCK] = total

    def place_body(q, _):
        rv = mly_r[pl.ds(q * LANES, LANES)]
        iv = mly_id[pl.ds(q * LANES, LANES)]
        rem = nrec - q * LANES
        bv = jnp.clip((lax.shift_right_logical(rv, 7) - lo) * 2
                      + jnp.minimum(iv & 3, 1), 0, NBUCK - 1)
        mask = lane_iota < rem
        pos = jnp.zeros((LANES,), jnp.int32)
        for lane in range(LANES):
            b = bv[lane]
            ps = cursor[b]
            pos = jnp.where(lane_iota == lane, ps, pos)
            @pl.when(rem > lane)
            def _():
                cursor[b] = ps + 1
        plsc.store_scatter(srt_r, [pos], rv, mask=mask)
        plsc.store_scatter(srt_id, [pos], iv, mask=mask)
        return ()
    lax.fori_loop(0, (nrec + LANES - 1) // LANES, place_body, ())

    rows_lo = lane_iota
    rows_hi = lane_iota + LANES

    def issue_slab(b):
        cnt = starts[b + 1] - starts[b]
        col = lo + lax.shift_right_logical(b, 1)
        c0 = pl.multiple_of(col * 128, 128)
        @pl.when((cnt > 0) & (b % 2 == 0))
        def _():
            pltpu.async_copy(u_tab.at[:, pl.ds(c0, 128)],
                             ring.at[b % DEPTH], sem_slab)
        @pl.when((cnt > 0) & (b % 2 == 1))
        def _():
            pltpu.async_copy(i_tab.at[:, pl.ds(c0, 128)],
                             ring.at[b % DEPTH], sem_slab)

    def drain_slab():
        pltpu.make_async_copy(u_tab.at[:, pl.ds(0, 128)],
                              ring.at[0], sem_slab).wait()

    def process_bucket(p, n_out):
        beg = starts[p]
        cnt = starts[p + 1] - beg
        slot = p % DEPTH

        def rec_body(q, n_out):
            off = beg + q * LANES
            rv = srt_r[pl.ds(off, LANES)]
            iv = srt_id[pl.ds(off, LANES)]
            rem = cnt - q * LANES
            for lane in range(LANES):
                valid = rem > lane
                so = n_out % STAG
                @pl.when(valid)
                def _():
                    c = jnp.full((LANES,), rv[lane] & 127, jnp.int32)
                    v0 = plsc.load_gather(ring.at[slot], [rows_lo, c])
                    v1 = plsc.load_gather(ring.at[slot], [rows_hi, c])
                    @pl.when(n_out >= STAG)
                    def _():
                        pltpu.make_async_copy(
                            u_emb.at[pl.ds(0, EMBED)], stag.at[0],
                            sem_out).wait()
                    stag[so, pl.ds(0, LANES)] = v0
                    stag[so, pl.ds(LANES, LANES)] = v1
                    kk = lax.shift_right_logical(iv[lane], 2)
                    tt = iv[lane] & 3
                    for t in range(3):
                        @pl.when(tt == t)
                        def _(t=t):
                            pltpu.async_copy(
                                stag.at[so],
                                embs[t].at[pl.ds(kk * EMBED, EMBED)],
                                sem_out)
                n_out = n_out + jnp.where(valid, 1, 0)
            return n_out

        nq = (cnt + LANES - 1) // LANES
        return lax.fori_loop(0, nq, rec_body, n_out)

    def maybe_process(p, n_out):
        @pl.when(starts[p + 1] - starts[p] > 0)
        def _():
            drain_slab()
        return lax.cond(starts[p + 1] - starts[p] > 0,
                        lambda n: process_bucket(p, n),
                        lambda n: n,
                        n_out)

    def bucket_body(b, n_out):
        issue_slab(b)
        p = b - (DEPTH - 1)
        return lax.cond(p >= 0,
                        lambda n: maybe_process(jnp.maximum(p, 0), n),
                        lambda n: n,
                        n_out)

    n_out = lax.fori_loop(0, NBUCK, bucket_body, jnp.int32(0))

    def tail_body(b0, n_out):
        return maybe_process(NBUCK - (DEPTH - 1) + b0, n_out)
    n_out = lax.fori_loop(0, DEPTH - 1, tail_body, n_out)

    def outdrain_body(i, _):
        pltpu.make_async_copy(u_emb.at[pl.ds(0, EMBED)], stag.at[0],
                              sem_out).wait()
        return ()
    lax.fori_loop(0, jnp.minimum(n_out, STAG), outdrain_body, ())


def _compute_body(u_emb, i_emb, n_emb, out_hbm, uv, iv, nv, out_v, sem):
    wid = lax.axis_index("s") * NUM_CORES + lax.axis_index("c")
    base = wid * BPW
    pltpu.sync_copy(u_emb.at[pl.ds(base * EMBED, BPW * EMBED)], uv)
    pltpu.sync_copy(i_emb.at[pl.ds(base * EMBED, BPW * EMBED)], iv)
    pltpu.sync_copy(n_emb.at[pl.ds(base * EMBED, BPW * EMBED)], nv)

    lane_iota = lax.iota(jnp.int32, LANES)

    def group_body(g, _):
        acc = jnp.zeros((LANES,), jnp.float32)
        for j in range(LANES):
            w = (g * LANES + j) * EMBED
            u0 = uv[pl.ds(w, LANES)]
            u1 = uv[pl.ds(w + LANES, LANES)]
            i0 = iv[pl.ds(w, LANES)]
            i1 = iv[pl.ds(w + LANES, LANES)]
            n0 = nv[pl.ds(w, LANES)]
            n1 = nv[pl.ds(w + LANES, LANES)]
            tv = u0 * (i0 - n0) + u1 * (i1 - n1)
            acc = jnp.where(lane_iota == j, jnp.sum(tv), acc)
        prob = 1.0 / (1.0 + jnp.exp(-acc))
        out_v[pl.ds(g * LANES, LANES)] = prob
        return ()

    lax.fori_loop(0, BPW // LANES, group_body, ())
    pltpu.sync_copy(out_v, out_hbm.at[pl.ds(base, BPW)])


@jax.jit
def kernel(user_table, item_table, user_tensor, item_tensor, nega_item_tensor):
    mesh = plsc.VectorSubcoreMesh(core_axis_name="c", subcore_axis_name="s")
    params = pltpu.CompilerParams(
        needs_layout_passes=False, use_tc_tiling_on_sc=True,
        disable_bounds_checks=True)

    extract = pl.kernel(
        _extract_body,
        out_type=(
            jax.ShapeDtypeStruct((BATCH * EMBED,), jnp.float32),
            jax.ShapeDtypeStruct((BATCH * EMBED,), jnp.float32),
            jax.ShapeDtypeStruct((BATCH * EMBED,), jnp.float32),
        ),
        mesh=mesh,
        scratch_types=[
            pltpu.VMEM((3 * BATCH,), jnp.int32),
            pltpu.VMEM((CAP,), jnp.int32),
            pltpu.VMEM((CAP,), jnp.int32),
            pltpu.VMEM((CAP,), jnp.int32),
            pltpu.VMEM((CAP,), jnp.int32),
            pltpu.VMEM((DEPTH, EMBED, 128), jnp.float32),
            pltpu.VMEM((STAG, EMBED), jnp.float32),
            pltpu.SMEM((NBUCK + 1,), jnp.int32),
            pltpu.SMEM((NBUCK,), jnp.int32),
            pltpu.SemaphoreType.DMA,
            pltpu.SemaphoreType.DMA,
        ],
        compiler_params=params,
    )
    compute = pl.kernel(
        _compute_body,
        out_type=jax.ShapeDtypeStruct((BATCH,), jnp.float32),
        mesh=mesh,
        scratch_types=[
            pltpu.VMEM((BPW * EMBED,), jnp.float32),
            pltpu.VMEM((BPW * EMBED,), jnp.float32),
            pltpu.VMEM((BPW * EMBED,), jnp.float32),
            pltpu.VMEM((BPW,), jnp.float32),
            pltpu.SemaphoreType.DMA,
        ],
        compiler_params=params,
    )

    u_emb, i_emb, n_emb = extract(
        user_table.T,
        item_table.T,
        user_tensor.astype(jnp.int32),
        item_tensor.astype(jnp.int32),
        nega_item_tensor.astype(jnp.int32),
    )
    return compute(u_emb, i_emb, n_emb)
